# baseline (speedup 1.0000x reference)
import functools

import jax
import jax.numpy as jnp
from jax import lax
from jax.experimental import pallas as pl
from jax.experimental.pallas import tpu as pltpu
from jax.experimental.pallas import tpu_sc as plsc

_N = 10000
_E = 160000
_LEAK = 0.1
_EPS = 1e-5

_NC = 2
_NS = 16
_NW = _NC * _NS
_EW = _E // _NW
_C = 1000
_NCH = _EW // _C

_BM = 1600
_BE = 1600


def _lr(z):
    return jnp.where(z >= 0, z, _LEAK * z)


def _estats_body(e_ref, s_ref, q_ref):
    i = pl.program_id(0)
    eb = e_ref[...]
    s = jnp.sum(eb, axis=0, keepdims=True)
    q = jnp.sum(eb * eb, axis=0, keepdims=True)

    @pl.when(i == 0)
    def _():
        s_ref[...] = s
        q_ref[...] = q

    @pl.when(i > 0)
    def _():
        s_ref[...] = s_ref[...] + s
        q_ref[...] = q_ref[...] + q


def _edge_stats(e):
    nb = 16
    blk = _E // nb
    s, q = pl.pallas_call(
        _estats_body,
        grid=(nb,),
        in_specs=[pl.BlockSpec((blk, 16), lambda i: (i, 0))],
        out_specs=[pl.BlockSpec((1, 16), lambda i: (0, 0)),
                   pl.BlockSpec((1, 16), lambda i: (0, 0))],
        out_shape=[jax.ShapeDtypeStruct((1, 16), jnp.float32),
                   jax.ShapeDtypeStruct((1, 16), jnp.float32)],
    )(e)
    return s, q


def _prep_body(x_ref, g_ref, b_ref, root_ref, bias_ref, h0_ref, xr0_ref):
    x = x_ref[...]
    m = jnp.mean(x, axis=0, keepdims=True)
    c = x - m
    v = jnp.mean(c * c, axis=0, keepdims=True)
    h0 = g_ref[...] * c * lax.rsqrt(v + _EPS) + b_ref[...]
    h0_ref[...] = h0
    xr0_ref[...] = (jnp.dot(h0, root_ref[...], preferred_element_type=jnp.float32)
                    + bias_ref[...])


def _prep(x, g, b, root0, bias0):
    return pl.pallas_call(
        _prep_body,
        out_shape=[jax.ShapeDtypeStruct((_N, 16), jnp.float32),
                   jax.ShapeDtypeStruct((_N, 32), jnp.float32)],
    )(x, g.reshape(1, 16), b.reshape(1, 16), root0, bias0.reshape(1, 32))


def _msg_body(e_ref, hs_ref, esc_ref, esh_ref, W0_ref, b0_ref, W1_ref, b1_ref,
              R_ref, S_ref, out_ref, *, fin):
    en = e_ref[...] * esc_ref[...] + esh_ref[...]
    t = _lr(jnp.dot(en.astype(jnp.bfloat16), W0_ref[...],
                    preferred_element_type=jnp.float32) + b0_ref[...])
    w = _lr(jnp.dot(t.astype(jnp.bfloat16), W1_ref[...],
                    preferred_element_type=jnp.float32) + b1_ref[...])
    hx = jnp.dot(hs_ref[...].astype(jnp.bfloat16), R_ref[...],
                 preferred_element_type=jnp.float32)
    q = w * hx
    q_hi = q.astype(jnp.bfloat16)
    q_lo = (q - q_hi.astype(jnp.float32)).astype(jnp.bfloat16)
    out_ref[...] = (jnp.dot(q_hi, S_ref[...], preferred_element_type=jnp.float32)
                    + jnp.dot(q_lo, S_ref[...], preferred_element_type=jnp.float32))


def _messages(e, hs, esc, esh, W0, b0, W1, b1, fin, start, n):
    hid = W0.shape[1]
    wdim = W1.shape[1]
    off = start // _BM
    R = jnp.kron(jnp.eye(fin, dtype=jnp.bfloat16), jnp.ones((1, 32), jnp.bfloat16))
    S = jnp.kron(jnp.ones((fin, 1), jnp.bfloat16), jnp.eye(32, dtype=jnp.bfloat16))
    S = jnp.pad(S, ((0, 0), (0, 96)))
    grid = (n // _BM,)
    return pl.pallas_call(
        functools.partial(_msg_body, fin=fin),
        grid=grid,
        in_specs=[
            pl.BlockSpec((_BM, 16), lambda i: (i + off, 0)),
            pl.BlockSpec((_BM, fin), lambda i: (i, 0)),
            pl.BlockSpec((1, 16), lambda i: (0, 0)),
            pl.BlockSpec((1, 16), lambda i: (0, 0)),
            pl.BlockSpec((16, hid), lambda i: (0, 0)),
            pl.BlockSpec((1, hid), lambda i: (0, 0)),
            pl.BlockSpec((hid, wdim), lambda i: (0, 0)),
            pl.BlockSpec((1, wdim), lambda i: (0, 0)),
            pl.BlockSpec((fin, wdim), lambda i: (0, 0)),
            pl.BlockSpec((wdim, 128), lambda i: (0, 0)),
        ],
        out_specs=pl.BlockSpec((_BM, 128), lambda i: (i, 0)),
        out_shape=jax.ShapeDtypeStruct((n, 128), jnp.float32),
    )(e, hs, esc, esh, W0.astype(jnp.bfloat16), b0.reshape(1, hid),
      W1.astype(jnp.bfloat16), b1.reshape(1, wdim), R, S)


def _hup_body(pa_ref, pb_ref, xr_ref, root_ref, bias_ref, h_ref, xr_out_ref):
    h = pa_ref[0] + pa_ref[1] + pb_ref[0] + pb_ref[1] + xr_ref[...]
    h_ref[...] = h
    xr_out_ref[...] = (jnp.dot(h, root_ref[...], preferred_element_type=jnp.float32)
                       + bias_ref[...])


def _h_update(parts_a, parts_b, xr, root, bias):
    return pl.pallas_call(
        _hup_body,
        out_shape=[jax.ShapeDtypeStruct((_N, 32), jnp.float32),
                   jax.ShapeDtypeStruct((_N, 32), jnp.float32)],
    )(parts_a, parts_b, xr, root, bias.reshape(1, 32))


def _hfin_body(pa_ref, pb_ref, xr_ref, h_ref):
    h_ref[...] = pa_ref[0] + pa_ref[1] + pb_ref[0] + pb_ref[1] + xr_ref[...]


def _h_final(parts_a, parts_b, xr):
    return pl.pallas_call(
        _hfin_body,
        out_shape=jax.ShapeDtypeStruct((_N, 32), jnp.float32),
    )(parts_a, parts_b, xr)


def _ep_body(hs_ref, hd_ref, e_ref, esc_ref, esh_ref,
             W0_ref, b0_ref, W1_ref, b1_ref, W2_ref, b2_ref, W3_ref, b3_ref,
             W4_ref, b4_ref, out_ref):
    en = e_ref[...] * esc_ref[...] + esh_ref[...]
    W0 = W0_ref[...]
    z = (jnp.dot(hs_ref[...], W0[0:32], preferred_element_type=jnp.float32)
         + jnp.dot(hd_ref[...], W0[32:64], preferred_element_type=jnp.float32)
         + jnp.dot(en, W0[64:80], preferred_element_type=jnp.float32)
         + b0_ref[...])
    z = _lr(z)
    z = _lr(jnp.dot(z, W1_ref[...], preferred_element_type=jnp.float32) + b1_ref[...])
    z = _lr(jnp.dot(z, W2_ref[...], preferred_element_type=jnp.float32) + b2_ref[...])
    z = _lr(jnp.dot(z, W3_ref[...], preferred_element_type=jnp.float32) + b3_ref[...])
    out_ref[...] = (jnp.dot(z, W4_ref[...], preferred_element_type=jnp.float32)
                    + b4_ref[...])


def _edge_pred(hs, hd, e, esc, esh, p, start, n):
    grid = (n // _BE,)
    off = start // _BE
    full = lambda r, c: pl.BlockSpec((r, c), lambda i: (0, 0))
    return pl.pallas_call(
        _ep_body,
        grid=grid,
        in_specs=[
            pl.BlockSpec((_BE, 32), lambda i: (i, 0)),
            pl.BlockSpec((_BE, 32), lambda i: (i, 0)),
            pl.BlockSpec((_BE, 16), lambda i: (i + off, 0)),
            full(1, 16), full(1, 16),
            full(80, 64), full(1, 64),
            full(64, 32), full(1, 32),
            full(32, 16), full(1, 16),
            full(16, 8), full(1, 8),
            full(8, 2), full(1, 2),
        ],
        out_specs=pl.BlockSpec((_BE, 2), lambda i: (i, 0)),
        out_shape=jax.ShapeDtypeStruct((n, 2), jnp.float32),
    )(hs, hd, e, esc, esh,
      p['ep_W0'], p['ep_b0'].reshape(1, 64),
      p['ep_W1'], p['ep_b1'].reshape(1, 32),
      p['ep_W2'], p['ep_b2'].reshape(1, 16),
      p['ep_W3'], p['ep_b3'].reshape(1, 8),
      p['ep_W4'], p['ep_b4'].reshape(1, 2))


def _sc_mesh():
    return plsc.VectorSubcoreMesh(core_axis_name="c", subcore_axis_name="s",
                                  num_cores=_NC, num_subcores=_NS)


def _gather(tab, idx, F, start, n):
    ew = n // _NW
    nch = ew // _C

    @functools.partial(
        pl.kernel,
        mesh=_sc_mesh(),
        out_type=jax.ShapeDtypeStruct((n, F), jnp.float32),
        scratch_types=[pltpu.VMEM((_C,), jnp.int32),
                       pltpu.VMEM((_C, F), jnp.float32),
                       pltpu.SemaphoreType.DMA],
        compiler_params=pltpu.CompilerParams(use_tc_tiling_on_sc=False),
    )
    def gk(tab_hbm, idx_hbm, out_hbm, idx_v, rows_v, sem):
        wid = lax.axis_index("s") * _NC + lax.axis_index("c")
        lbase = pl.multiple_of(wid * ew, 8)
        for j in range(nch):
            loff = lbase + j * _C
            pltpu.sync_copy(idx_hbm.at[pl.ds(start + loff, _C)], idx_v)
            pltpu.async_copy(tab_hbm.at[idx_v], rows_v, sem).wait()
            pltpu.sync_copy(rows_v, out_hbm.at[pl.ds(loff, _C)])

    return gk(tab, idx)


def _gather2(tab, idx2, start, n):
    ew = n // _NW
    nch = ew // _C

    @functools.partial(
        pl.kernel,
        mesh=_sc_mesh(),
        out_type=[jax.ShapeDtypeStruct((n, 32), jnp.float32),
                  jax.ShapeDtypeStruct((n, 32), jnp.float32)],
        scratch_types=[pltpu.VMEM((_C,), jnp.int32),
                       pltpu.VMEM((_C, 32), jnp.float32),
                       pltpu.SemaphoreType.DMA],
        compiler_params=pltpu.CompilerParams(use_tc_tiling_on_sc=False),
    )
    def gk(tab_hbm, idx_hbm, out0_hbm, out1_hbm, idx_v, rows_v, sem):
        wid = lax.axis_index("s") * _NC + lax.axis_index("c")
        lbase = pl.multiple_of(wid * ew, 8)
        for r, out_hbm in ((0, out0_hbm), (1, out1_hbm)):
            for j in range(nch):
                loff = lbase + j * _C
                pltpu.sync_copy(idx_hbm.at[r, pl.ds(start + loff, _C)], idx_v)
                pltpu.async_copy(tab_hbm.at[idx_v], rows_v, sem).wait()
                pltpu.sync_copy(rows_v, out_hbm.at[pl.ds(loff, _C)])

    return gk(tab, idx2)


def _scatter_add(msg, dst, init2, start, n):
    ew = n // _NW
    nch = ew // _C

    @functools.partial(
        pl.kernel,
        mesh=_sc_mesh(),
        out_type=jax.ShapeDtypeStruct((_NC, _N, 32), jnp.float32),
        scratch_types=[pltpu.VMEM((_C,), jnp.int32),
                       pltpu.VMEM((_C, 32), jnp.float32),
                       pltpu.VMEM_SHARED((_N, 32), jnp.float32)],
        compiler_params=pltpu.CompilerParams(use_tc_tiling_on_sc=False),
    )
    def sk(msg_hbm, dst_hbm, init_hbm, out_hbm, idx_v, msg_v, acc_sh):
        cid = lax.axis_index("c")
        sid = lax.axis_index("s")

        @pl.when(sid == 0)
        def _():
            pltpu.sync_copy(init_hbm.at[cid], acc_sh)

        plsc.subcore_barrier()
        wid = sid * _NC + cid
        lbase = pl.multiple_of(wid * ew, 8)
        for j in range(nch):
            loff = lbase + j * _C
            pltpu.sync_copy(dst_hbm.at[pl.ds(start + loff, _C)], idx_v)
            pltpu.sync_copy(msg_hbm.at[pl.ds(loff, _C), pl.ds(0, 32)], msg_v)
            pltpu.sync_copy(msg_v, acc_sh.at[idx_v], add=True)
        plsc.subcore_barrier()
        rows = _N // _NS
        rb = sid * rows
        pltpu.sync_copy(acc_sh.at[pl.ds(rb, rows)],
                        out_hbm.at[cid, pl.ds(rb, rows)])

    return sk(msg, dst, init2)


def kernel(x, e, params, edge_index, xbatch):
    p = params
    src = edge_index[0]
    dst = edge_index[1]
    zeros2 = jnp.zeros((_NC, _N, 32), jnp.float32)

    s, q = _edge_stats(e)
    m = s / _E
    esc = p['bn_edge_g'].reshape(1, 16) * lax.rsqrt(q / _E - m * m + _EPS)
    esh = p['bn_edge_b'].reshape(1, 16) - m * esc

    h0, xr0 = _prep(x, p['bn_node_g'], p['bn_node_b'], p['root0'], p['bias0'])

    splits = ((0, 96000), (96000, 64000))

    hs0 = [_gather(h0, src, 16, s, n) for s, n in splits]
    msg0 = [_messages(e, hs0[k], esc, esh, p['nn0_W0'], p['nn0_b0'],
                      p['nn0_W1'], p['nn0_b1'], 16, s, n)
            for k, (s, n) in enumerate(splits)]
    parts0 = [_scatter_add(msg0[k], dst, zeros2, s, n)
              for k, (s, n) in enumerate(splits)]
    h1, xr1 = _h_update(parts0[0], parts0[1], xr0, p['root1'], p['bias1'])

    hs1 = [_gather(h1, src, 32, s, n) for s, n in splits]
    msg1 = [_messages(e, hs1[k], esc, esh, p['nn1_W0'], p['nn1_b0'],
                      p['nn1_W1'], p['nn1_b1'], 32, s, n)
            for k, (s, n) in enumerate(splits)]
    parts1 = [_scatter_add(msg1[k], dst, zeros2, s, n)
              for k, (s, n) in enumerate(splits)]
    h2 = _h_final(parts1[0], parts1[1], xr1)

    g2 = [_gather2(h2, edge_index, s, n) for s, n in splits]
    z = [_edge_pred(g2[k][0], g2[k][1], e, esc, esh, p, s, n)
         for k, (s, n) in enumerate(splits)]
    return jnp.concatenate(z, axis=0)


# speedup vs baseline: 1.2763x; 1.2763x over previous
import functools

import jax
import jax.numpy as jnp
from jax import lax
from jax.experimental import pallas as pl
from jax.experimental.pallas import tpu as pltpu
from jax.experimental.pallas import tpu_sc as plsc

_N = 10000
_E = 160000
_LEAK = 0.1
_EPS = 1e-5

_NC = 2
_NS = 16
_NW = _NC * _NS
_EW = _E // _NW
_C = 1000
_NCH = _EW // _C

_BM = 1600
_BE = 1600


def _lr(z):
    return jnp.where(z >= 0, z, _LEAK * z)


def _estats_body(e_ref, s_ref, q_ref):
    i = pl.program_id(0)
    eb = e_ref[...]
    s = jnp.sum(eb, axis=0, keepdims=True)
    q = jnp.sum(eb * eb, axis=0, keepdims=True)

    @pl.when(i == 0)
    def _():
        s_ref[...] = s
        q_ref[...] = q

    @pl.when(i > 0)
    def _():
        s_ref[...] = s_ref[...] + s
        q_ref[...] = q_ref[...] + q


def _edge_stats(e):
    nb = 16
    blk = _E // nb
    s, q = pl.pallas_call(
        _estats_body,
        grid=(nb,),
        in_specs=[pl.BlockSpec((blk, 16), lambda i: (i, 0))],
        out_specs=[pl.BlockSpec((1, 16), lambda i: (0, 0)),
                   pl.BlockSpec((1, 16), lambda i: (0, 0))],
        out_shape=[jax.ShapeDtypeStruct((1, 16), jnp.float32),
                   jax.ShapeDtypeStruct((1, 16), jnp.float32)],
    )(e)
    return s, q


def _prep_body(x_ref, g_ref, b_ref, root_ref, bias_ref, h0_ref, xr0_ref):
    x = x_ref[...]
    m = jnp.mean(x, axis=0, keepdims=True)
    c = x - m
    v = jnp.mean(c * c, axis=0, keepdims=True)
    h0 = g_ref[...] * c * lax.rsqrt(v + _EPS) + b_ref[...]
    h0_ref[...] = h0
    xr0_ref[...] = (jnp.dot(h0, root_ref[...], preferred_element_type=jnp.float32)
                    + bias_ref[...])


def _prep(x, g, b, root0, bias0):
    return pl.pallas_call(
        _prep_body,
        out_shape=[jax.ShapeDtypeStruct((_N, 16), jnp.float32),
                   jax.ShapeDtypeStruct((_N, 32), jnp.float32)],
    )(x, g.reshape(1, 16), b.reshape(1, 16), root0, bias0.reshape(1, 32))


def _msg_body(e_ref, hs_ref, esc_ref, esh_ref, W0_ref, b0_ref, W1_ref, b1_ref,
              R_ref, S_ref, out_ref, *, fin):
    en = e_ref[...] * esc_ref[...] + esh_ref[...]
    t = _lr(jnp.dot(en.astype(jnp.bfloat16), W0_ref[...],
                    preferred_element_type=jnp.float32) + b0_ref[...])
    w = _lr(jnp.dot(t.astype(jnp.bfloat16), W1_ref[...],
                    preferred_element_type=jnp.float32) + b1_ref[...])
    hx = jnp.dot(hs_ref[...].astype(jnp.bfloat16), R_ref[...],
                 preferred_element_type=jnp.float32)
    q = (w * hx).astype(jnp.bfloat16)
    out_ref[...] = jnp.dot(q, S_ref[...], preferred_element_type=jnp.float32)


def _messages(e, hs, esc, esh, W0, b0, W1, b1, fin, start, n):
    hid = W0.shape[1]
    wdim = W1.shape[1]
    off = start // _BM
    R = jnp.kron(jnp.eye(fin, dtype=jnp.bfloat16), jnp.ones((1, 32), jnp.bfloat16))
    S = jnp.kron(jnp.ones((fin, 1), jnp.bfloat16), jnp.eye(32, dtype=jnp.bfloat16))
    S = jnp.pad(S, ((0, 0), (0, 96)))
    grid = (n // _BM,)
    return pl.pallas_call(
        functools.partial(_msg_body, fin=fin),
        grid=grid,
        in_specs=[
            pl.BlockSpec((_BM, 16), lambda i: (i + off, 0)),
            pl.BlockSpec((_BM, fin), lambda i: (i, 0)),
            pl.BlockSpec((1, 16), lambda i: (0, 0)),
            pl.BlockSpec((1, 16), lambda i: (0, 0)),
            pl.BlockSpec((16, hid), lambda i: (0, 0)),
            pl.BlockSpec((1, hid), lambda i: (0, 0)),
            pl.BlockSpec((hid, wdim), lambda i: (0, 0)),
            pl.BlockSpec((1, wdim), lambda i: (0, 0)),
            pl.BlockSpec((fin, wdim), lambda i: (0, 0)),
            pl.BlockSpec((wdim, 128), lambda i: (0, 0)),
        ],
        out_specs=pl.BlockSpec((_BM, 128), lambda i: (i, 0)),
        out_shape=jax.ShapeDtypeStruct((n, 128), jnp.float32),
    )(e, hs, esc, esh, W0.astype(jnp.bfloat16), b0.reshape(1, hid),
      W1.astype(jnp.bfloat16), b1.reshape(1, wdim), R, S)


def _hup_body(pa_ref, pb_ref, xr_ref, root_ref, bias_ref, h_ref, xr_out_ref):
    h = pa_ref[0] + pa_ref[1] + pb_ref[0] + pb_ref[1] + xr_ref[...]
    h_ref[...] = h
    xr_out_ref[...] = (jnp.dot(h, root_ref[...], preferred_element_type=jnp.float32)
                       + bias_ref[...])


def _h_update(parts_a, parts_b, xr, root, bias):
    return pl.pallas_call(
        _hup_body,
        out_shape=[jax.ShapeDtypeStruct((_N, 32), jnp.float32),
                   jax.ShapeDtypeStruct((_N, 32), jnp.float32)],
    )(parts_a, parts_b, xr, root, bias.reshape(1, 32))


def _hfin_body(pa_ref, pb_ref, xr_ref, h_ref):
    h_ref[...] = pa_ref[0] + pa_ref[1] + pb_ref[0] + pb_ref[1] + xr_ref[...]


def _h_final(parts_a, parts_b, xr):
    return pl.pallas_call(
        _hfin_body,
        out_shape=jax.ShapeDtypeStruct((_N, 32), jnp.float32),
    )(parts_a, parts_b, xr)


def _ep_body(hs_ref, hd_ref, e_ref, esc_ref, esh_ref,
             W0_ref, b0_ref, W1_ref, b1_ref, W2_ref, b2_ref, W3_ref, b3_ref,
             W4_ref, b4_ref, out_ref):
    en = e_ref[...] * esc_ref[...] + esh_ref[...]
    W0 = W0_ref[...]
    z = (jnp.dot(hs_ref[...], W0[0:32], preferred_element_type=jnp.float32)
         + jnp.dot(hd_ref[...], W0[32:64], preferred_element_type=jnp.float32)
         + jnp.dot(en, W0[64:80], preferred_element_type=jnp.float32)
         + b0_ref[...])
    z = _lr(z)
    z = _lr(jnp.dot(z, W1_ref[...], preferred_element_type=jnp.float32) + b1_ref[...])
    z = _lr(jnp.dot(z, W2_ref[...], preferred_element_type=jnp.float32) + b2_ref[...])
    z = _lr(jnp.dot(z, W3_ref[...], preferred_element_type=jnp.float32) + b3_ref[...])
    out_ref[...] = (jnp.dot(z, W4_ref[...], preferred_element_type=jnp.float32)
                    + b4_ref[...])


def _edge_pred(hs, hd, e, esc, esh, p, start, n):
    grid = (n // _BE,)
    off = start // _BE
    full = lambda r, c: pl.BlockSpec((r, c), lambda i: (0, 0))
    return pl.pallas_call(
        _ep_body,
        grid=grid,
        in_specs=[
            pl.BlockSpec((_BE, 32), lambda i: (i, 0)),
            pl.BlockSpec((_BE, 32), lambda i: (i, 0)),
            pl.BlockSpec((_BE, 16), lambda i: (i + off, 0)),
            full(1, 16), full(1, 16),
            full(80, 64), full(1, 64),
            full(64, 32), full(1, 32),
            full(32, 16), full(1, 16),
            full(16, 8), full(1, 8),
            full(8, 2), full(1, 2),
        ],
        out_specs=pl.BlockSpec((_BE, 2), lambda i: (i, 0)),
        out_shape=jax.ShapeDtypeStruct((n, 2), jnp.float32),
    )(hs, hd, e, esc, esh,
      p['ep_W0'], p['ep_b0'].reshape(1, 64),
      p['ep_W1'], p['ep_b1'].reshape(1, 32),
      p['ep_W2'], p['ep_b2'].reshape(1, 16),
      p['ep_W3'], p['ep_b3'].reshape(1, 8),
      p['ep_W4'], p['ep_b4'].reshape(1, 2))


def _sc_mesh():
    return plsc.VectorSubcoreMesh(core_axis_name="c", subcore_axis_name="s",
                                  num_cores=_NC, num_subcores=_NS)


def _gather(tab, idx, F, start, n):
    ew = n // _NW
    nch = ew // _C

    @functools.partial(
        pl.kernel,
        mesh=_sc_mesh(),
        out_type=jax.ShapeDtypeStruct((n, F), jnp.float32),
        scratch_types=[pltpu.VMEM((_C,), jnp.int32),
                       pltpu.VMEM((_C, F), jnp.float32),
                       pltpu.SemaphoreType.DMA],
        compiler_params=pltpu.CompilerParams(use_tc_tiling_on_sc=False),
    )
    def gk(tab_hbm, idx_hbm, out_hbm, idx_v, rows_v, sem):
        wid = lax.axis_index("s") * _NC + lax.axis_index("c")
        lbase = pl.multiple_of(wid * ew, 8)
        for j in range(nch):
            loff = lbase + j * _C
            pltpu.sync_copy(idx_hbm.at[pl.ds(start + loff, _C)], idx_v)
            pltpu.async_copy(tab_hbm.at[idx_v], rows_v, sem).wait()
            pltpu.sync_copy(rows_v, out_hbm.at[pl.ds(loff, _C)])

    return gk(tab, idx)


def _gather2(tab, idx2, start, n):
    ew = n // _NW
    nch = ew // _C

    @functools.partial(
        pl.kernel,
        mesh=_sc_mesh(),
        out_type=[jax.ShapeDtypeStruct((n, 32), jnp.float32),
                  jax.ShapeDtypeStruct((n, 32), jnp.float32)],
        scratch_types=[pltpu.VMEM((_C,), jnp.int32),
                       pltpu.VMEM((_C, 32), jnp.float32),
                       pltpu.SemaphoreType.DMA],
        compiler_params=pltpu.CompilerParams(use_tc_tiling_on_sc=False),
    )
    def gk(tab_hbm, idx_hbm, out0_hbm, out1_hbm, idx_v, rows_v, sem):
        wid = lax.axis_index("s") * _NC + lax.axis_index("c")
        lbase = pl.multiple_of(wid * ew, 8)
        for r, out_hbm in ((0, out0_hbm), (1, out1_hbm)):
            for j in range(nch):
                loff = lbase + j * _C
                pltpu.sync_copy(idx_hbm.at[r, pl.ds(start + loff, _C)], idx_v)
                pltpu.async_copy(tab_hbm.at[idx_v], rows_v, sem).wait()
                pltpu.sync_copy(rows_v, out_hbm.at[pl.ds(loff, _C)])

    return gk(tab, idx2)


def _scatter_add(msg, dst, init2, start, n):
    ew = n // _NW
    nch = ew // _C

    @functools.partial(
        pl.kernel,
        mesh=_sc_mesh(),
        out_type=jax.ShapeDtypeStruct((_NC, _N, 32), jnp.float32),
        scratch_types=[pltpu.VMEM((_C,), jnp.int32),
                       pltpu.VMEM((_C, 32), jnp.float32),
                       pltpu.VMEM_SHARED((_N, 32), jnp.float32)],
        compiler_params=pltpu.CompilerParams(use_tc_tiling_on_sc=False),
    )
    def sk(msg_hbm, dst_hbm, init_hbm, out_hbm, idx_v, msg_v, acc_sh):
        cid = lax.axis_index("c")
        sid = lax.axis_index("s")

        @pl.when(sid == 0)
        def _():
            pltpu.sync_copy(init_hbm.at[cid], acc_sh)

        plsc.subcore_barrier()
        wid = sid * _NC + cid
        lbase = pl.multiple_of(wid * ew, 8)
        for j in range(nch):
            loff = lbase + j * _C
            pltpu.sync_copy(dst_hbm.at[pl.ds(start + loff, _C)], idx_v)
            pltpu.sync_copy(msg_hbm.at[pl.ds(loff, _C), pl.ds(0, 32)], msg_v)
            pltpu.sync_copy(msg_v, acc_sh.at[idx_v], add=True)
        plsc.subcore_barrier()
        rows = _N // _NS
        rb = sid * rows
        pltpu.sync_copy(acc_sh.at[pl.ds(rb, rows)],
                        out_hbm.at[cid, pl.ds(rb, rows)])

    return sk(msg, dst, init2)


def kernel(x, e, params, edge_index, xbatch):
    p = params
    src = edge_index[0]
    dst = edge_index[1]
    zeros2 = jnp.zeros((_NC, _N, 32), jnp.float32)

    s, q = _edge_stats(e)
    m = s / _E
    esc = p['bn_edge_g'].reshape(1, 16) * lax.rsqrt(q / _E - m * m + _EPS)
    esh = p['bn_edge_b'].reshape(1, 16) - m * esc

    h0, xr0 = _prep(x, p['bn_node_g'], p['bn_node_b'], p['root0'], p['bias0'])

    splits = ((0, 96000), (96000, 64000))

    hs0 = [_gather(h0, src, 16, s, n) for s, n in splits]
    msg0 = [_messages(e, hs0[k], esc, esh, p['nn0_W0'], p['nn0_b0'],
                      p['nn0_W1'], p['nn0_b1'], 16, s, n)
            for k, (s, n) in enumerate(splits)]
    parts0 = [_scatter_add(msg0[k], dst, zeros2, s, n)
              for k, (s, n) in enumerate(splits)]
    h1, xr1 = _h_update(parts0[0], parts0[1], xr0, p['root1'], p['bias1'])

    hs1 = [_gather(h1, src, 32, s, n) for s, n in splits]
    msg1 = [_messages(e, hs1[k], esc, esh, p['nn1_W0'], p['nn1_b0'],
                      p['nn1_W1'], p['nn1_b1'], 32, s, n)
            for k, (s, n) in enumerate(splits)]
    parts1 = [_scatter_add(msg1[k], dst, zeros2, s, n)
              for k, (s, n) in enumerate(splits)]
    h2 = _h_final(parts1[0], parts1[1], xr1)

    g2 = [_gather2(h2, edge_index, s, n) for s, n in splits]
    z = [_edge_pred(g2[k][0], g2[k][1], e, esc, esh, p, s, n)
         for k, (s, n) in enumerate(splits)]
    return jnp.concatenate(z, axis=0)
